# Initial kernel scaffold; baseline (speedup 1.0000x reference)
import jax
import jax.numpy as jnp
from jax.experimental import pallas as pl


def kernel(x, edge_index, W1, b1, W2, b2, W3, b3, Wpg, bpg, Wpd, bpd, Wv, bv):
    raise NotImplementedError("write your pallas kernel here")


# speedup vs baseline: 2.3010x; 2.3010x over previous
import functools

import jax
import jax.numpy as jnp
from jax import lax
from jax.experimental import pallas as pl
from jax.experimental.pallas import tpu as pltpu
from jax.experimental.pallas import tpu_sc as plsc

_N = 10000
_NP = 10240
_E = 320000
_EP = 327680
_D = 128
_NW = 32
_EPW = _EP // _NW
_K = 128
_NCHUNK = _EPW // _K
_RPT = _NP // 16

@functools.cache
def _sc_agg_kernel():
    mesh = plsc.VectorSubcoreMesh(core_axis_name="c", subcore_axis_name="s")

    @functools.partial(
        pl.kernel,
        mesh=mesh,
        out_type=jax.ShapeDtypeStruct((2, _NP, _D), jnp.float32),
        scratch_types=[
            pltpu.VMEM((_K,), jnp.int32),
            pltpu.VMEM((_K,), jnp.int32),
            pltpu.VMEM((_K, _D), jnp.float32),
            pltpu.VMEM_SHARED((_NP, _D), jnp.float32),
            pltpu.SemaphoreType.DMA,
        ],
    )
    def _sc_agg(h_hbm, src_hbm, dst_hbm, zeros_hbm, out_hbm,
                idx_s, idx_d, rows, agg_sh, sem):
        c = lax.axis_index("c")
        s = lax.axis_index("s")
        wid = s * 2 + c
        pltpu.sync_copy(zeros_hbm.at[pl.ds(s * _RPT, _RPT)],
                        agg_sh.at[pl.ds(s * _RPT, _RPT)])
        plsc.subcore_barrier()
        base = wid * _EPW

        def body(i, carry):
            off = base + i * _K
            pltpu.sync_copy(src_hbm.at[pl.ds(off, _K)], idx_s)
            pltpu.sync_copy(dst_hbm.at[pl.ds(off, _K)], idx_d)
            pltpu.async_copy(h_hbm.at[idx_s], rows, sem).wait()
            pltpu.sync_copy(rows, agg_sh.at[idx_d], add=True)
            return carry

        lax.fori_loop(0, _NCHUNK, body, 0)
        plsc.subcore_barrier()
        pltpu.sync_copy(agg_sh.at[pl.ds(s * _RPT, _RPT)],
                        out_hbm.at[c, pl.ds(s * _RPT, _RPT)])

    return _sc_agg


def _lin_body(p_ref, W_ref, b_ref, o_ref):
    acc = p_ref[0] + p_ref[1]
    o_ref[...] = jnp.maximum(
        jnp.dot(acc, W_ref[...], preferred_element_type=jnp.float32)
        + b_ref[...], 0.0)


def _linear_relu(p, W, b2d):
    blk = 1024
    return pl.pallas_call(
        _lin_body,
        grid=(_NP // blk,),
        in_specs=[
            pl.BlockSpec((2, blk, _D), lambda i: (0, i, 0)),
            pl.BlockSpec((_D, _D), lambda i: (0, 0)),
            pl.BlockSpec((1, _D), lambda i: (0, 0)),
        ],
        out_specs=pl.BlockSpec((blk, _D), lambda i: (i, 0)),
        out_shape=jax.ShapeDtypeStruct((_NP, _D), jnp.float32),
    )(p, W, b2d)


def _heads_body(h_ref, Wpg_ref, bpg_ref, Wpd_ref, bpd_ref, Wv_ref, bv_ref,
                pi_ref, pid_ref, v_ref):
    h = h_ref[...]
    rid = lax.broadcasted_iota(jnp.int32, (_NP, _D), 0)
    hm = jnp.where(rid < _N, h, 0.0)
    mN = jnp.sum(hm, axis=0, keepdims=True) * (1.0 / _N)
    pi_ref[...] = (jnp.dot(h, Wpg_ref[...], preferred_element_type=jnp.float32)
                   + bpg_ref[...])
    pid_ref[...] = (jnp.dot(mN, Wpd_ref[...], preferred_element_type=jnp.float32)
                    + bpd_ref[...])
    v_ref[...] = (jnp.dot(mN, Wv_ref[...], preferred_element_type=jnp.float32)
                  + bv_ref[...])


def _heads(h, Wpg, bpg, Wpd, bpd, Wv, bv):
    return pl.pallas_call(
        _heads_body,
        out_shape=[
            jax.ShapeDtypeStruct((_NP, 1), jnp.float32),
            jax.ShapeDtypeStruct((1, 1), jnp.float32),
            jax.ShapeDtypeStruct((1, 1), jnp.float32),
        ],
    )(h, Wpg, bpg.reshape(1, 1), Wpd, bpd.reshape(1, 1), Wv, bv.reshape(1, 1))


def kernel(x, edge_index, W1, b1, W2, b2, W3, b3, Wpg, bpg, Wpd, bpd, Wv, bv):
    src = edge_index[0]
    dst = edge_index[1]
    pad_e = _EP - _E
    src_p = jnp.concatenate([src, jnp.zeros((pad_e,), jnp.int32)])
    dst_p = jnp.concatenate([dst, jnp.full((pad_e,), _N, jnp.int32)])
    h = jnp.concatenate([x, jnp.zeros((_NP - _N, _D), jnp.float32)], axis=0)
    zeros = jnp.zeros((_NP, _D), jnp.float32)
    for W, b in ((W1, b1), (W2, b2), (W3, b3)):
        p = _sc_agg_kernel()(h, src_p, dst_p, zeros)
        h = _linear_relu(p, W, b.reshape(1, _D))
    pi_nodes, pi_done, v = _heads(h, Wpg, bpg, Wpd, bpd, Wv, bv)
    pi = jnp.concatenate([pi_nodes[:_N], pi_done], axis=0)
    return (pi, v)


# speedup vs baseline: 2.5214x; 1.0958x over previous
import functools

import jax
import jax.numpy as jnp
from jax import lax
from jax.experimental import pallas as pl
from jax.experimental.pallas import tpu as pltpu
from jax.experimental.pallas import tpu_sc as plsc

_N = 10000
_NP = 10240
_E = 320000
_EP = 327680
_D = 128
_NW = 32
_EPW = _EP // _NW
_K = 80
_NCHUNK = _EPW // _K
_RPT = _NP // 16

@functools.cache
def _sc_agg_kernel():
    mesh = plsc.VectorSubcoreMesh(core_axis_name="c", subcore_axis_name="s")

    @functools.partial(
        pl.kernel,
        mesh=mesh,
        out_type=jax.ShapeDtypeStruct((2, _NP, _D), jnp.float32),
        scratch_types=[
            pltpu.VMEM((2, 2, _K), jnp.int32),
            pltpu.VMEM((_K, _D), jnp.float32),
            pltpu.VMEM((_K, _D), jnp.float32),
            pltpu.VMEM_SHARED((_NP, _D), jnp.float32),
            pltpu.SemaphoreType.DMA,
            pltpu.SemaphoreType.DMA,
            pltpu.SemaphoreType.DMA,
            pltpu.SemaphoreType.DMA,
        ],
    )
    def _sc_agg(h_hbm, idx_hbm, zeros_hbm, out_hbm,
                idx_v, rows0, rows1, agg_sh, rs0, rs1, is0, is1):
        c = lax.axis_index("c")
        s = lax.axis_index("s")
        wid = s * 2 + c
        pltpu.sync_copy(zeros_hbm.at[pl.ds(s * _RPT, _RPT)],
                        agg_sh.at[pl.ds(s * _RPT, _RPT)])
        plsc.subcore_barrier()
        bufs = (rows0, rows1)
        rsems = (rs0, rs1)
        isems = (is0, is1)
        pltpu.async_copy(idx_hbm.at[wid, 0], idx_v.at[0], isems[0]).wait()
        pltpu.async_copy(h_hbm.at[idx_v.at[0, 0]], rows0, rsems[0])
        pltpu.async_copy(idx_hbm.at[wid, 1], idx_v.at[1], isems[1])

        def body(j0, carry):
            for b in range(2):
                j = j0 + b
                nb = 1 - b
                pltpu.make_async_copy(idx_hbm.at[wid, j + 1], idx_v.at[nb],
                                      isems[nb]).wait()
                pltpu.make_async_copy(h_hbm.at[idx_v.at[b, 0]], bufs[b],
                                      rsems[b]).wait()
                pltpu.async_copy(h_hbm.at[idx_v.at[nb, 0]], bufs[nb],
                                 rsems[nb])
                pltpu.sync_copy(bufs[b], agg_sh.at[idx_v.at[b, 1]], add=True)
                pltpu.async_copy(idx_hbm.at[wid, j + 2], idx_v.at[b],
                                 isems[b])
            return carry

        lax.fori_loop(0, (_NCHUNK - 2) // 2, lambda i, car: body(i * 2, car),
                      0, unroll=False)
        for j in (_NCHUNK - 2, _NCHUNK - 1):
            b = j % 2
            pltpu.make_async_copy(h_hbm.at[idx_v.at[b, 0]], bufs[b],
                                  rsems[b]).wait()
            if j + 1 < _NCHUNK:
                pltpu.make_async_copy(idx_hbm.at[wid, j + 1],
                                      idx_v.at[1 - b], isems[1 - b]).wait()
                pltpu.async_copy(h_hbm.at[idx_v.at[1 - b, 0]], bufs[1 - b],
                                 rsems[1 - b])
            pltpu.sync_copy(bufs[b], agg_sh.at[idx_v.at[b, 1]], add=True)
        plsc.subcore_barrier()
        pltpu.sync_copy(agg_sh.at[pl.ds(s * _RPT, _RPT)],
                        out_hbm.at[c, pl.ds(s * _RPT, _RPT)])

    return _sc_agg


def _lin_body(p_ref, W_ref, b_ref, o_ref):
    acc = p_ref[0] + p_ref[1]
    o_ref[...] = jnp.maximum(
        jnp.dot(acc, W_ref[...], preferred_element_type=jnp.float32)
        + b_ref[...], 0.0)


def _linear_relu(p, W, b2d):
    blk = 1024
    return pl.pallas_call(
        _lin_body,
        grid=(_NP // blk,),
        in_specs=[
            pl.BlockSpec((2, blk, _D), lambda i: (0, i, 0)),
            pl.BlockSpec((_D, _D), lambda i: (0, 0)),
            pl.BlockSpec((1, _D), lambda i: (0, 0)),
        ],
        out_specs=pl.BlockSpec((blk, _D), lambda i: (i, 0)),
        out_shape=jax.ShapeDtypeStruct((_NP, _D), jnp.float32),
    )(p, W, b2d)


def _heads_body(h_ref, Wpg_ref, bpg_ref, Wpd_ref, bpd_ref, Wv_ref, bv_ref,
                pi_ref, pid_ref, v_ref):
    h = h_ref[...]
    rid = lax.broadcasted_iota(jnp.int32, (_NP, _D), 0)
    hm = jnp.where(rid < _N, h, 0.0)
    mN = jnp.sum(hm, axis=0, keepdims=True) * (1.0 / _N)
    pi_ref[...] = (jnp.dot(h, Wpg_ref[...], preferred_element_type=jnp.float32)
                   + bpg_ref[...])
    pid_ref[...] = (jnp.dot(mN, Wpd_ref[...], preferred_element_type=jnp.float32)
                    + bpd_ref[...])
    v_ref[...] = (jnp.dot(mN, Wv_ref[...], preferred_element_type=jnp.float32)
                  + bv_ref[...])


def _heads(h, Wpg, bpg, Wpd, bpd, Wv, bv):
    return pl.pallas_call(
        _heads_body,
        out_shape=[
            jax.ShapeDtypeStruct((_NP, 1), jnp.float32),
            jax.ShapeDtypeStruct((1, 1), jnp.float32),
            jax.ShapeDtypeStruct((1, 1), jnp.float32),
        ],
    )(h, Wpg, bpg.reshape(1, 1), Wpd, bpd.reshape(1, 1), Wv, bv.reshape(1, 1))


def kernel(x, edge_index, W1, b1, W2, b2, W3, b3, Wpg, bpg, Wpd, bpd, Wv, bv):
    src = edge_index[0]
    dst = edge_index[1]
    pad_e = _EP - _E
    src_p = jnp.concatenate(
        [src, jnp.zeros((pad_e,), jnp.int32)]).reshape(_NW, _NCHUNK, 1, _K)
    dst_p = jnp.concatenate(
        [dst, jnp.full((pad_e,), _N, jnp.int32)]).reshape(_NW, _NCHUNK, 1, _K)
    idx_p = jnp.concatenate([src_p, dst_p], axis=2)
    h = jnp.concatenate([x, jnp.zeros((_NP - _N, _D), jnp.float32)], axis=0)
    zeros = jnp.zeros((_NP, _D), jnp.float32)
    for W, b in ((W1, b1), (W2, b2), (W3, b3)):
        p = _sc_agg_kernel()(h, idx_p, zeros)
        h = _linear_relu(p, W, b.reshape(1, _D))
    pi_nodes, pi_done, v = _heads(h, Wpg, bpg, Wpd, bpd, Wv, bv)
    pi = jnp.concatenate([pi_nodes[:_N], pi_done], axis=0)
    return (pi, v)


# speedup vs baseline: 2.8842x; 1.1439x over previous
import functools

import jax
import jax.numpy as jnp
from jax import lax
from jax.experimental import pallas as pl
from jax.experimental.pallas import tpu as pltpu
from jax.experimental.pallas import tpu_sc as plsc

_N = 10000
_NP = 10240
_E = 320000
_EP = 327680
_D = 128
_NW = 32
_EPW = _EP // _NW
_K = 80
_NCHUNK = _EPW // _K
_TCHUNK = _EP // _K
_CA = 204
_CB = 52
_RPT = _NP // 16

@functools.cache
def _sc_agg_kernel():
    mesh = plsc.VectorSubcoreMesh(core_axis_name="c", subcore_axis_name="s")

    @functools.partial(
        pl.kernel,
        mesh=mesh,
        out_type=jax.ShapeDtypeStruct((2, _NP, _D), jnp.float32),
        scratch_types=[
            pltpu.VMEM((2, 2, _K), jnp.int32),
            pltpu.VMEM((_K, _D), jnp.float32),
            pltpu.VMEM((_K, _D), jnp.float32),
            pltpu.VMEM_SHARED((_NP, _D), jnp.float32),
            pltpu.SemaphoreType.DMA,
            pltpu.SemaphoreType.DMA,
            pltpu.SemaphoreType.DMA,
            pltpu.SemaphoreType.DMA,
        ],
    )
    def _sc_agg(h_hbm, idx_hbm, zeros_hbm, out_hbm,
                idx_v, rows0, rows1, agg_sh, rs0, rs1, is0, is1):
        c = lax.axis_index("c")
        s = lax.axis_index("s")
        start = jnp.where(c == 0, s * _CA, 16 * _CA + s * _CB)
        n = jnp.where(c == 0, _CA, _CB)
        pltpu.sync_copy(zeros_hbm.at[pl.ds(s * _RPT, _RPT)],
                        agg_sh.at[pl.ds(s * _RPT, _RPT)])
        plsc.subcore_barrier()
        bufs = (rows0, rows1)
        rsems = (rs0, rs1)
        isems = (is0, is1)
        pltpu.async_copy(idx_hbm.at[start], idx_v.at[0], isems[0]).wait()
        pltpu.async_copy(h_hbm.at[idx_v.at[0, 0]], rows0, rsems[0])
        pltpu.async_copy(idx_hbm.at[start + 1], idx_v.at[1], isems[1])

        def body(j0, carry):
            for b in range(2):
                j = j0 + b
                nb = 1 - b
                pltpu.make_async_copy(idx_hbm.at[j + 1], idx_v.at[nb],
                                      isems[nb]).wait()
                pltpu.make_async_copy(h_hbm.at[idx_v.at[b, 0]], bufs[b],
                                      rsems[b]).wait()
                pltpu.async_copy(h_hbm.at[idx_v.at[nb, 0]], bufs[nb],
                                 rsems[nb])
                pltpu.sync_copy(bufs[b], agg_sh.at[idx_v.at[b, 1]], add=True)
                pltpu.async_copy(idx_hbm.at[j + 2], idx_v.at[b], isems[b])
            return carry

        lax.fori_loop(0, (n - 2) // 2,
                      lambda i, car: body(start + i * 2, car),
                      0, unroll=False)
        for jo in (0, 1):
            j = start + n - 2 + jo
            b = jo
            pltpu.make_async_copy(h_hbm.at[idx_v.at[b, 0]], bufs[b],
                                  rsems[b]).wait()
            if jo == 0:
                pltpu.make_async_copy(idx_hbm.at[j + 1],
                                      idx_v.at[1 - b], isems[1 - b]).wait()
                pltpu.async_copy(h_hbm.at[idx_v.at[1 - b, 0]], bufs[1 - b],
                                 rsems[1 - b])
            pltpu.sync_copy(bufs[b], agg_sh.at[idx_v.at[b, 1]], add=True)
        plsc.subcore_barrier()
        pltpu.sync_copy(agg_sh.at[pl.ds(s * _RPT, _RPT)],
                        out_hbm.at[c, pl.ds(s * _RPT, _RPT)])

    return _sc_agg


def _lin_body(p_ref, W_ref, b_ref, o_ref):
    acc = p_ref[0] + p_ref[1]
    o_ref[...] = jnp.maximum(
        jnp.dot(acc, W_ref[...], preferred_element_type=jnp.float32)
        + b_ref[...], 0.0)


def _linear_relu(p, W, b2d):
    blk = 1024
    return pl.pallas_call(
        _lin_body,
        grid=(_NP // blk,),
        in_specs=[
            pl.BlockSpec((2, blk, _D), lambda i: (0, i, 0)),
            pl.BlockSpec((_D, _D), lambda i: (0, 0)),
            pl.BlockSpec((1, _D), lambda i: (0, 0)),
        ],
        out_specs=pl.BlockSpec((blk, _D), lambda i: (i, 0)),
        out_shape=jax.ShapeDtypeStruct((_NP, _D), jnp.float32),
    )(p, W, b2d)


def _heads_body(h_ref, Wpg_ref, bpg_ref, Wpd_ref, bpd_ref, Wv_ref, bv_ref,
                pi_ref, pid_ref, v_ref):
    h = h_ref[...]
    rid = lax.broadcasted_iota(jnp.int32, (_NP, _D), 0)
    hm = jnp.where(rid < _N, h, 0.0)
    mN = jnp.sum(hm, axis=0, keepdims=True) * (1.0 / _N)
    pi_ref[...] = (jnp.dot(h, Wpg_ref[...], preferred_element_type=jnp.float32)
                   + bpg_ref[...])
    pid_ref[...] = (jnp.dot(mN, Wpd_ref[...], preferred_element_type=jnp.float32)
                    + bpd_ref[...])
    v_ref[...] = (jnp.dot(mN, Wv_ref[...], preferred_element_type=jnp.float32)
                  + bv_ref[...])


def _heads(h, Wpg, bpg, Wpd, bpd, Wv, bv):
    return pl.pallas_call(
        _heads_body,
        out_shape=[
            jax.ShapeDtypeStruct((_NP, 1), jnp.float32),
            jax.ShapeDtypeStruct((1, 1), jnp.float32),
            jax.ShapeDtypeStruct((1, 1), jnp.float32),
        ],
    )(h, Wpg, bpg.reshape(1, 1), Wpd, bpd.reshape(1, 1), Wv, bv.reshape(1, 1))


def kernel(x, edge_index, W1, b1, W2, b2, W3, b3, Wpg, bpg, Wpd, bpd, Wv, bv):
    src = edge_index[0]
    dst = edge_index[1]
    pad_e = _EP - _E
    src_p = jnp.concatenate(
        [src, jnp.zeros((pad_e,), jnp.int32)]).reshape(_TCHUNK, 1, _K)
    dst_p = jnp.concatenate(
        [dst, jnp.full((pad_e,), _N, jnp.int32)]).reshape(_TCHUNK, 1, _K)
    idx_p = jnp.concatenate([src_p, dst_p], axis=1)
    h = jnp.concatenate([x, jnp.zeros((_NP - _N, _D), jnp.float32)], axis=0)
    zeros = jnp.zeros((_NP, _D), jnp.float32)
    for W, b in ((W1, b1), (W2, b2), (W3, b3)):
        p = _sc_agg_kernel()(h, idx_p, zeros)
        h = _linear_relu(p, W, b.reshape(1, _D))
    pi_nodes, pi_done, v = _heads(h, Wpg, bpg, Wpd, bpd, Wv, bv)
    pi = jnp.concatenate([pi_nodes[:_N], pi_done], axis=0)
    return (pi, v)


# speedup vs baseline: 2.9050x; 1.0072x over previous
import functools

import jax
import jax.numpy as jnp
from jax import lax
from jax.experimental import pallas as pl
from jax.experimental.pallas import tpu as pltpu
from jax.experimental.pallas import tpu_sc as plsc

_N = 10000
_NP = 10240
_E = 320000
_EP = 327680
_D = 128
_NW = 32
_EPW = _EP // _NW
_K = 80
_NCHUNK = _EPW // _K
_TCHUNK = _EP // _K
_CA = 204
_CB = 52
_RPT = _NP // 16

@functools.cache
def _sc_agg_kernel():
    mesh = plsc.VectorSubcoreMesh(core_axis_name="c", subcore_axis_name="s")

    @functools.partial(
        pl.kernel,
        mesh=mesh,
        out_type=jax.ShapeDtypeStruct((2, _NP, _D), jnp.float32),
        scratch_types=[
            pltpu.VMEM((2, 2, _K), jnp.int32),
            pltpu.VMEM((_K, _D), jnp.float32),
            pltpu.VMEM((_K, _D), jnp.float32),
            pltpu.VMEM_SHARED((_NP, _D), jnp.float32),
            pltpu.SemaphoreType.DMA,
            pltpu.SemaphoreType.DMA,
            pltpu.SemaphoreType.DMA,
            pltpu.SemaphoreType.DMA,
        ],
    )
    def _sc_agg(h_hbm, idx_hbm, out_hbm,
                idx_v, rows0, rows1, agg_sh, rs0, rs1, is0, is1):
        c = lax.axis_index("c")
        s = lax.axis_index("s")
        start = jnp.where(c == 0, s * _CA, 16 * _CA + s * _CB)
        n = jnp.where(c == 0, _CA, _CB)
        zv = jnp.zeros((16,), jnp.float32)

        def zbody(i, carry):
            for jj in range(8):
                rows0[i, pl.ds(jj * 16, 16)] = zv
            return carry

        lax.fori_loop(0, _K, zbody, 0, unroll=False)
        for r in range(_RPT // _K):
            pltpu.sync_copy(rows0, agg_sh.at[pl.ds(s * _RPT + r * _K, _K)])
        plsc.subcore_barrier()
        bufs = (rows0, rows1)
        rsems = (rs0, rs1)
        isems = (is0, is1)
        pltpu.async_copy(idx_hbm.at[start], idx_v.at[0], isems[0]).wait()
        pltpu.async_copy(h_hbm.at[idx_v.at[0, 0]], rows0, rsems[0])
        pltpu.async_copy(idx_hbm.at[start + 1], idx_v.at[1], isems[1])

        def body(j0, carry):
            for b in range(2):
                j = j0 + b
                nb = 1 - b
                pltpu.make_async_copy(idx_hbm.at[j + 1], idx_v.at[nb],
                                      isems[nb]).wait()
                pltpu.make_async_copy(h_hbm.at[idx_v.at[b, 0]], bufs[b],
                                      rsems[b]).wait()
                pltpu.async_copy(h_hbm.at[idx_v.at[nb, 0]], bufs[nb],
                                 rsems[nb])
                pltpu.sync_copy(bufs[b], agg_sh.at[idx_v.at[b, 1]], add=True)
                pltpu.async_copy(idx_hbm.at[j + 2], idx_v.at[b], isems[b])
            return carry

        lax.fori_loop(0, (n - 2) // 2,
                      lambda i, car: body(start + i * 2, car),
                      0, unroll=False)
        for jo in (0, 1):
            j = start + n - 2 + jo
            b = jo
            pltpu.make_async_copy(h_hbm.at[idx_v.at[b, 0]], bufs[b],
                                  rsems[b]).wait()
            if jo == 0:
                pltpu.make_async_copy(idx_hbm.at[j + 1],
                                      idx_v.at[1 - b], isems[1 - b]).wait()
                pltpu.async_copy(h_hbm.at[idx_v.at[1 - b, 0]], bufs[1 - b],
                                 rsems[1 - b])
            pltpu.sync_copy(bufs[b], agg_sh.at[idx_v.at[b, 1]], add=True)
        plsc.subcore_barrier()
        pltpu.sync_copy(agg_sh.at[pl.ds(s * _RPT, _RPT)],
                        out_hbm.at[c, pl.ds(s * _RPT, _RPT)])

    return _sc_agg


def _lin_body(p_ref, W_ref, b_ref, o_ref):
    acc = p_ref[0] + p_ref[1]
    o_ref[...] = jnp.maximum(
        jnp.dot(acc, W_ref[...], preferred_element_type=jnp.float32)
        + b_ref[...], 0.0)


def _linear_relu(p, W, b2d):
    blk = 1024
    return pl.pallas_call(
        _lin_body,
        grid=(_NP // blk,),
        in_specs=[
            pl.BlockSpec((2, blk, _D), lambda i: (0, i, 0)),
            pl.BlockSpec((_D, _D), lambda i: (0, 0)),
            pl.BlockSpec((1, _D), lambda i: (0, 0)),
        ],
        out_specs=pl.BlockSpec((blk, _D), lambda i: (i, 0)),
        out_shape=jax.ShapeDtypeStruct((_NP, _D), jnp.float32),
    )(p, W, b2d)


def _heads_body(h_ref, Wpg_ref, bpg_ref, Wpd_ref, bpd_ref, Wv_ref, bv_ref,
                pi_ref, pid_ref, v_ref):
    h = h_ref[...]
    rid = lax.broadcasted_iota(jnp.int32, (_NP, _D), 0)
    hm = jnp.where(rid < _N, h, 0.0)
    mN = jnp.sum(hm, axis=0, keepdims=True) * (1.0 / _N)
    pi_ref[...] = (jnp.dot(h, Wpg_ref[...], preferred_element_type=jnp.float32)
                   + bpg_ref[...])
    pid_ref[...] = (jnp.dot(mN, Wpd_ref[...], preferred_element_type=jnp.float32)
                    + bpd_ref[...])
    v_ref[...] = (jnp.dot(mN, Wv_ref[...], preferred_element_type=jnp.float32)
                  + bv_ref[...])


def _heads(h, Wpg, bpg, Wpd, bpd, Wv, bv):
    return pl.pallas_call(
        _heads_body,
        out_shape=[
            jax.ShapeDtypeStruct((_NP, 1), jnp.float32),
            jax.ShapeDtypeStruct((1, 1), jnp.float32),
            jax.ShapeDtypeStruct((1, 1), jnp.float32),
        ],
    )(h, Wpg, bpg.reshape(1, 1), Wpd, bpd.reshape(1, 1), Wv, bv.reshape(1, 1))


def kernel(x, edge_index, W1, b1, W2, b2, W3, b3, Wpg, bpg, Wpd, bpd, Wv, bv):
    src = edge_index[0]
    dst = edge_index[1]
    pad_e = _EP - _E
    src_p = jnp.concatenate(
        [src, jnp.zeros((pad_e,), jnp.int32)]).reshape(_TCHUNK, 1, _K)
    dst_p = jnp.concatenate(
        [dst, jnp.full((pad_e,), _N, jnp.int32)]).reshape(_TCHUNK, 1, _K)
    idx_p = jnp.concatenate([src_p, dst_p], axis=1)
    h = jnp.concatenate([x, jnp.zeros((_NP - _N, _D), jnp.float32)], axis=0)
    for W, b in ((W1, b1), (W2, b2), (W3, b3)):
        p = _sc_agg_kernel()(h, idx_p)
        h = _linear_relu(p, W, b.reshape(1, _D))
    pi_nodes, pi_done, v = _heads(h, Wpg, bpg, Wpd, bpd, Wv, bv)
    pi = jnp.concatenate([pi_nodes[:_N], pi_done], axis=0)
    return (pi, v)
